# baseline (speedup 1.0000x reference)
import jax
import jax.numpy as jnp
from jax import lax
from jax.experimental import pallas as pl
from jax.experimental.pallas import tpu as pltpu, tpu_sc as plsc

_N_VAL = 1000
_ROWS = 4096 * 20
_NW = 32
_RPW = _ROWS // _NW
_R = 64
_NCHUNK = _RPW // _R


def _sc_body(x_hbm, zeros_hbm, out_hbm, idx_v, buf_v):
    wid = lax.axis_index("s") * 2 + lax.axis_index("c")
    base_row = wid * _RPW
    pltpu.sync_copy(x_hbm.at[pl.ds(base_row, _RPW)], idx_v)
    pltpu.sync_copy(zeros_hbm, buf_v)

    ones16 = jnp.full((16,), 1.0, jnp.float32)
    zeros16 = jnp.zeros((16,), jnp.float32)
    lane = lax.iota(jnp.int32, 16)

    def chunk_body(c, carry):
        for j in range(_R // 16):
            xv = idx_v[pl.ds(c * _R + j * 16, 16)]
            pos = (lane + j * 16) * _N_VAL + xv
            plsc.store_scatter(buf_v, [pos], ones16)
        pltpu.sync_copy(
            buf_v,
            out_hbm.at[pl.ds((base_row + c * _R) * _N_VAL, _R * _N_VAL)],
        )
        for j in range(_R // 16):
            xv = idx_v[pl.ds(c * _R + j * 16, 16)]
            pos = (lane + j * 16) * _N_VAL + xv
            plsc.store_scatter(buf_v, [pos], zeros16)
        return carry

    lax.fori_loop(0, _NCHUNK, chunk_body, 0)


def kernel(x):
    xf = x.reshape(-1)
    zeros = jnp.zeros((_R * _N_VAL,), jnp.float32)
    mesh = plsc.VectorSubcoreMesh(core_axis_name="c", subcore_axis_name="s")
    out = pl.kernel(
        _sc_body,
        out_type=jax.ShapeDtypeStruct((_ROWS * _N_VAL,), jnp.float32),
        mesh=mesh,
        scratch_types=[
            pltpu.VMEM((_RPW,), jnp.int32),
            pltpu.VMEM((_R * _N_VAL,), jnp.float32),
        ],
        compiler_params=pltpu.CompilerParams(needs_layout_passes=False),
    )(xf, zeros)
    return out.reshape(x.shape + (_N_VAL,))
